# baseline (speedup 1.0000x reference)
import functools

import jax
import jax.numpy as jnp
from jax import lax
from jax.experimental import pallas as pl
from jax.experimental.pallas import tpu as pltpu
from jax.experimental.pallas import tpu_sc as plsc

SCORE_THRESH = 0.05
NMS_THRESH = 0.45
TOP_N = 100
N = 20000
NPAD = 20480
NCLS = 20
NL = 112
NCH = 1280
NC2 = 80


def _prep_body(priors_ref, deltas_ref, obj_ref, sc_ref, bx_ref, mc_ref):
    pcx = priors_ref[0:1, :]
    pcy = priors_ref[1:2, :]
    pw = priors_ref[2:3, :] + 1e-3
    ph = priors_ref[3:4, :] + 1e-3
    d0 = deltas_ref[0, 0:1, :]
    d1 = deltas_ref[0, 1:2, :]
    d2 = deltas_ref[0, 2:3, :]
    d3 = deltas_ref[0, 3:4, :]
    cx = pcx + d0 * 0.1 * pw
    cy = pcy + d1 * 0.1 * ph
    w = pw * jnp.exp(d2 * 0.2)
    h = ph * jnp.exp(d3 * 0.2)
    x1 = cx - w * 0.5
    y1 = cy - h * 0.5
    x2 = cx + w * 0.5
    y2 = cy + h * 0.5

    n_row = lax.broadcasted_iota(jnp.int32, (1, NPAD), 1)
    vrow = n_row < N
    neg = jnp.float32(-1e30)
    mx = jnp.maximum(
        jnp.maximum(jnp.max(jnp.where(vrow, x1, neg)), jnp.max(jnp.where(vrow, y1, neg))),
        jnp.maximum(jnp.max(jnp.where(vrow, x2, neg)), jnp.max(jnp.where(vrow, y2, neg))),
    )
    mc = mx + 1.0

    o = obj_ref[0]
    m0 = jnp.max(o, axis=0, keepdims=True)
    e = jnp.exp(o - m0)
    p = e / jnp.sum(e, axis=0, keepdims=True)
    S = p[1:21, :]
    valid2d = lax.broadcasted_iota(jnp.int32, (NCLS, NPAD), 1) < N
    sc_ref[0] = jnp.where(valid2d & (S > SCORE_THRESH), S, -1.0)

    bx_ref[0, 0:1, :] = x1
    bx_ref[0, 1:2, :] = y1
    bx_ref[0, 2:3, :] = x2
    bx_ref[0, 3:4, :] = y2
    mc_ref[0] = jnp.full((8, 128), mc, jnp.float32)


def _tr_body(o_ref, ot_ref):
    ot_ref[0] = o_ref[0].T


def _tr(objectness):
    B = objectness.shape[0]
    nb = 8
    blk = NPAD // nb
    return pl.pallas_call(
        _tr_body,
        grid=(B, nb),
        in_specs=[pl.BlockSpec((1, blk, 21), lambda b, t: (b, t, 0))],
        out_specs=pl.BlockSpec((1, 21, blk), lambda b, t: (b, 0, t)),
        out_shape=jax.ShapeDtypeStruct((B, 21, NPAD), jnp.float32),
    )(objectness)


def _prep(priors, pred_bbox_deltas, objectness):
    B = pred_bbox_deltas.shape[0]
    pr_t = jnp.pad(priors, ((0, NPAD - N), (0, 0))).T
    dl_t = jnp.pad(pred_bbox_deltas, ((0, 0), (0, NPAD - N), (0, 0))).transpose(0, 2, 1)
    ob_t = _tr(objectness)
    return pl.pallas_call(
        _prep_body,
        grid=(B,),
        in_specs=[
            pl.BlockSpec((4, NPAD), lambda b: (0, 0)),
            pl.BlockSpec((1, 4, NPAD), lambda b: (b, 0, 0)),
            pl.BlockSpec((1, 21, NPAD), lambda b: (b, 0, 0)),
        ],
        out_specs=[
            pl.BlockSpec((1, NCLS, NPAD), lambda b: (b, 0, 0)),
            pl.BlockSpec((1, 4, NPAD), lambda b: (b, 0, 0)),
            pl.BlockSpec((1, 8, 128), lambda b: (b, 0, 0)),
        ],
        out_shape=[
            jax.ShapeDtypeStruct((B, NCLS, NPAD), jnp.float32),
            jax.ShapeDtypeStruct((B, 4, NPAD), jnp.float32),
            jax.ShapeDtypeStruct((B, 8, 128), jnp.float32),
        ],
    )(pr_t, dl_t, ob_t)


def _lane():
    return lax.broadcasted_iota(jnp.int32, (16,), 0)


def _sget(ref, i):
    v = plsc.load_gather(ref, [jnp.full((16,), i, jnp.int32)])
    return jnp.max(v)


def _sgetv(ref, i):
    return plsc.load_gather(ref, [jnp.full((16,), i, jnp.int32)])


def _sget2(ref, i, j):
    v = plsc.load_gather(ref, [jnp.full((16,), i, jnp.int32),
                               jnp.full((16,), j, jnp.int32)])
    return jnp.max(v)


def _set1(ref, i, val):
    base = (i // 16) * 16
    v = ref[pl.ds(base, 16)]
    ref[pl.ds(base, 16)] = jnp.where(_lane() == i - base, val, v)


def _ffs(mask):
    return jnp.max(plsc.all_reduce_ffs(mask))


def _popcnt(mask):
    return jnp.max(plsc.all_reduce_population_count(mask))


def _make_sc_body(cap):
  nchk = (cap + 15) // 16

  def _sc_nms_body(sc_hbm, bx_hbm, mc_hbm, ob_hbm, os_hbm, ol_hbm, ls_hbm, ln_hbm,
                 fl_hbm,
                 S_v, bx_v, L1_v, L2_v, ax1_v, ay1_v, ax2_v, ay2_v, aa_v,
                 ps_v, pn_v, mc_v, fl_v, sa_v, na_v, hd_v, obo_v, oso_v, olo_v):
    f32 = jnp.float32
    cc = lax.axis_index("c")
    ss = lax.axis_index("s")
    bb = ss // 8
    b = cc * 2 + bb
    g = ss % 8

    for k in range(4):
        pltpu.sync_copy(bx_hbm.at[b, k], bx_v.at[pl.ds(k * NPAD, NPAD)])
    pltpu.sync_copy(mc_hbm.at[b, 0], mc_v)
    mc_s = jnp.max(mc_v[pl.ds(0, 16)])
    lane = _lane()

    for slot in range(3):
        for k in range(7):
            ps_v[pl.ds(k * 16, 16)] = jnp.full((16,), -1.0, f32)
            pn_v[pl.ds(k * 16, 16)] = jnp.zeros((16,), jnp.int32)
        valid = (g < 4) | (jnp.int32(slot) < 2)
        cls = jnp.where(g < 4, 3 * g + slot, 12 + 2 * (g - 4) + slot)

        @pl.when(valid)
        def _walk():
            pltpu.sync_copy(sc_hbm.at[b, cls], S_v)
            for k in range(nchk):
                ax1_v[pl.ds(k * 16, 16)] = jnp.full((16,), 1e30, f32)
                ay1_v[pl.ds(k * 16, 16)] = jnp.full((16,), 1e30, f32)
                ax2_v[pl.ds(k * 16, 16)] = jnp.full((16,), 1e30, f32)
                ay2_v[pl.ds(k * 16, 16)] = jnp.full((16,), 1e30, f32)
                aa_v[pl.ds(k * 16, 16)] = jnp.zeros((16,), f32)

            def l1body(t, c):
                base = t * 256
                m = plsc.load_gather(S_v, [lane * 16 + base])
                for l in range(1, 16):
                    m = jnp.maximum(m, plsc.load_gather(S_v, [lane * 16 + (base + l)]))
                L1_v[pl.ds(t * 16, 16)] = m
                return c
            lax.fori_loop(0, NCH // 16, l1body, 0)

            def l2body(t, c):
                base = t * 256
                m = plsc.load_gather(L1_v, [lane * 16 + base])
                for l in range(1, 16):
                    m = jnp.maximum(m, plsc.load_gather(L1_v, [lane * 16 + (base + l)]))
                L2_v[pl.ds(t * 16, 16)] = m
                return c
            lax.fori_loop(0, NC2 // 16, l2body, 0)

            r = L2_v[pl.ds(0, 16)]
            for q in range(1, 5):
                r = jnp.maximum(r, L2_v[pl.ds(q * 16, 16)])
            root0 = jnp.max(r)
            lmc = (cls + 1).astype(f32) * mc_s

            def wcond(carry):
                cnt, pops, fails, root = carry
                return (cnt < cap) & (root > 0.0) & (pops < NPAD) & (fails == 0)

            def wbody(carry):
                cnt, pops, fails, root = carry
                q = jnp.int32(-1)
                for qq in range(5):
                    v = L2_v[pl.ds(qq * 16, 16)]
                    mk = v == root
                    hit = _popcnt(mk) > 0
                    q = jnp.where((q < 0) & hit, qq * 16 + _ffs(mk), q)
                fails = fails + jnp.where(q < 0, 1, 0)
                q = jnp.clip(q, 0, NC2 - 1)
                l1c = L1_v[pl.ds(q * 16, 16)]
                jj = _ffs(l1c == root)
                fails = fails + jnp.where(jj > 15, 1, 0)
                jj = jnp.clip(jj, 0, 15)
                j = q * 16 + jj
                sc_ = S_v[pl.ds(j * 16, 16)]
                ll = _ffs(sc_ == root)
                fails = fails + jnp.where(ll > 15, 1, 0)
                ll = jnp.clip(ll, 0, 15)
                n = j * 16 + ll
                ox1 = _sgetv(bx_v, n) + lmc
                oy1 = _sgetv(bx_v, NPAD + n) + lmc
                ox2 = _sgetv(bx_v, 2 * NPAD + n) + lmc
                oy2 = _sgetv(bx_v, 3 * NPAD + n) + lmc
                a2 = (ox2 - ox1) * (oy2 - oy1)

                rej = jnp.int32(0)
                for k in range(nchk):
                    x1a = ax1_v[pl.ds(k * 16, 16)]
                    y1a = ay1_v[pl.ds(k * 16, 16)]
                    x2a = ax2_v[pl.ds(k * 16, 16)]
                    y2a = ay2_v[pl.ds(k * 16, 16)]
                    aa = aa_v[pl.ds(k * 16, 16)]
                    iw = jnp.maximum(jnp.minimum(x2a, ox2) - jnp.maximum(x1a, ox1), 0.0)
                    ih = jnp.maximum(jnp.minimum(y2a, oy2) - jnp.maximum(y1a, oy1), 0.0)
                    inter = iw * ih
                    iou = inter / (aa + a2 - inter + 1e-9)
                    rej = rej + _popcnt(iou > NMS_THRESH)
                acc = rej == 0

                @pl.when(acc)
                def _accept():
                    _set1(ax1_v, cnt, ox1)
                    _set1(ay1_v, cnt, oy1)
                    _set1(ax2_v, cnt, ox2)
                    _set1(ay2_v, cnt, oy2)
                    _set1(aa_v, cnt, a2)
                    _set1(ps_v, cnt, root)
                    _set1(pn_v, cnt, n)

                sc2 = jnp.where(lane == ll, -1.0, sc_)
                S_v[pl.ds(j * 16, 16)] = sc2
                l1n = jnp.max(sc2)
                l1c2 = jnp.where(lane == jj, l1n, l1c)
                L1_v[pl.ds(q * 16, 16)] = l1c2
                qq16 = (q // 16) * 16
                l2c = L2_v[pl.ds(qq16, 16)]
                l2c2 = jnp.where(lane == q - qq16, jnp.max(l1c2), l2c)
                L2_v[pl.ds(qq16, 16)] = l2c2
                r = L2_v[pl.ds(0, 16)]
                for qq in range(1, 5):
                    r = jnp.maximum(r, L2_v[pl.ds(qq * 16, 16)])
                return (jnp.where(acc, cnt + 1, cnt), pops + 1, fails, jnp.max(r))

            cntf, _, _, rootf = lax.while_loop(
                wcond, wbody, (jnp.int32(0), jnp.int32(0), jnp.int32(0), root0))

            @pl.when((cntf == cap) & (rootf > 0.0))
            def _mark_truncated():
                _set1(ps_v, cap, jnp.float32(-2.0))

        li = g * 3 + slot
        pltpu.sync_copy(ps_v, ls_hbm.at[b, li])
        pltpu.sync_copy(pn_v, ln_hbm.at[b, li])

    plsc.subcore_barrier()

    @pl.when(g == 0)
    def _merge():
        for l in range(24, 32):
            for k in range(7):
                sa_v[l, pl.ds(k * 16, 16)] = jnp.full((16,), -1.0, f32)
        pltpu.sync_copy(ls_hbm.at[b, pl.ds(0, 24)], sa_v.at[pl.ds(0, 24)])
        pltpu.sync_copy(ln_hbm.at[b, pl.ds(0, 24)], na_v.at[pl.ds(0, 24)])
        hd_v[pl.ds(0, 16)] = jnp.zeros((16,), jnp.int32)
        hd_v[pl.ds(16, 16)] = jnp.zeros((16,), jnp.int32)
        for k in range(7):
            oso_v[pl.ds(k * 16, 16)] = jnp.zeros((16,), f32)
            olo_v[pl.ds(k * 16, 16)] = jnp.zeros((16,), jnp.int32)
        for k in range(28):
            obo_v[pl.ds(k * 16, 16)] = jnp.zeros((16,), f32)

        def heads_root():
            h1 = hd_v[pl.ds(0, 16)]
            h2 = hd_v[pl.ds(16, 16)]
            sv1 = plsc.load_gather(sa_v, [lane, h1])
            sv2 = plsc.load_gather(sa_v, [lane + 16, h2])
            return sv1, sv2, jnp.max(jnp.maximum(sv1, sv2))

        _, _, root0 = heads_root()

        def mcond(carry):
            t, root = carry
            return (t < TOP_N) & (root > 0.0)

        def mbody(carry):
            t, root = carry
            h1 = hd_v[pl.ds(0, 16)]
            h2 = hd_v[pl.ds(16, 16)]
            sv1 = plsc.load_gather(sa_v, [lane, h1])
            sv2 = plsc.load_gather(sa_v, [lane + 16, h2])
            mk1 = sv1 == root
            lsel = jnp.where(_popcnt(mk1) > 0, _ffs(mk1),
                             16 + jnp.clip(_ffs(sv2 == root), 0, 15))
            lsel = jnp.clip(lsel, 0, 31)
            h = jnp.max(jnp.maximum(jnp.where(lane == lsel, h1, 0),
                                    jnp.where(lane + 16 == lsel, h2, 0)))
            n = _sget2(na_v, lsel, h)
            hd_v[pl.ds(0, 16)] = jnp.where(lane == lsel, h + 1, h1)
            hd_v[pl.ds(16, 16)] = jnp.where(lane + 16 == lsel, h + 1, h2)
            g2 = lsel // 3
            sl = lsel - g2 * 3
            clsm = jnp.where(g2 < 4, 3 * g2 + sl, 12 + 2 * (g2 - 4) + sl)
            _set1(oso_v, t, root)
            _set1(olo_v, t, clsm + 1)
            for k in range(4):
                v = _sget(bx_v, k * NPAD + n)
                _set1(obo_v, k * NL + t, jnp.minimum(jnp.maximum(v, 0.0), 1.0))
            h1 = hd_v[pl.ds(0, 16)]
            h2 = hd_v[pl.ds(16, 16)]
            sv1 = plsc.load_gather(sa_v, [lane, h1])
            sv2 = plsc.load_gather(sa_v, [lane + 16, h2])
            return t + 1, jnp.max(jnp.maximum(sv1, sv2))

        lax.while_loop(mcond, mbody, (jnp.int32(0), root0))
        h1 = hd_v[pl.ds(0, 16)]
        h2 = hd_v[pl.ds(16, 16)]
        sv1 = plsc.load_gather(sa_v, [lane, h1])
        sv2 = plsc.load_gather(sa_v, [lane + 16, h2])
        vio = _popcnt(sv1 == -2.0) + _popcnt(sv2 == -2.0)
        fl_v[pl.ds(0, 16)] = jnp.where(lane == 0, vio, 0)
        pltpu.sync_copy(fl_v, fl_hbm.at[b])
        pltpu.sync_copy(obo_v, ob_hbm.at[b])
        pltpu.sync_copy(oso_v, os_hbm.at[b])
        pltpu.sync_copy(olo_v, ol_hbm.at[b])

  return _sc_nms_body


def _sc_nms(scores, boxes, mcs, cap):
    B = scores.shape[0]
    f32, i32 = jnp.float32, jnp.int32
    mesh = plsc.VectorSubcoreMesh(core_axis_name="c", subcore_axis_name="s")
    fn = functools.partial(
        pl.kernel, mesh=mesh,
        compiler_params=pltpu.CompilerParams(needs_layout_passes=False),
        out_type=[
            jax.ShapeDtypeStruct((B, 4 * NL), f32),
            jax.ShapeDtypeStruct((B, NL), f32),
            jax.ShapeDtypeStruct((B, NL), i32),
            jax.ShapeDtypeStruct((B, 32, NL), f32),
            jax.ShapeDtypeStruct((B, 32, NL), i32),
            jax.ShapeDtypeStruct((B, 16), i32),
        ],
        scratch_types=[
            pltpu.VMEM((NPAD,), f32),
            pltpu.VMEM((4 * NPAD,), f32),
            pltpu.VMEM((NCH,), f32),
            pltpu.VMEM((NC2,), f32),
            pltpu.VMEM((NL,), f32),
            pltpu.VMEM((NL,), f32),
            pltpu.VMEM((NL,), f32),
            pltpu.VMEM((NL,), f32),
            pltpu.VMEM((NL,), f32),
            pltpu.VMEM((NL,), f32),
            pltpu.VMEM((NL,), i32),
            pltpu.VMEM((128,), f32),
            pltpu.VMEM((16,), i32),
            pltpu.VMEM((32, NL), f32),
            pltpu.VMEM((32, NL), i32),
            pltpu.VMEM((32,), i32),
            pltpu.VMEM((4 * NL,), f32),
            pltpu.VMEM((NL,), f32),
            pltpu.VMEM((NL,), i32),
        ],
    )(_make_sc_body(cap))
    return fn(scores, boxes, mcs)


FAST_CAP = 32


def kernel(priors, pred_bbox_deltas, objectness):
    B = pred_bbox_deltas.shape[0]
    scores, boxes, mcs = _prep(priors, pred_bbox_deltas, objectness)
    fast = _sc_nms(scores, boxes, mcs, FAST_CAP)
    ok = jnp.sum(fast[5]) == 0

    def _use_fast(ops):
        return ops[0][:3]

    def _full(ops):
        return _sc_nms(*ops[1], TOP_N)[:3]

    ob, osc, ol = lax.cond(ok, _use_fast, _full, (fast, (scores, boxes, mcs)))
    boxes_out = ob.reshape(B, 4, NL)[:, :, :TOP_N].transpose(0, 2, 1)
    return boxes_out, osc[:, :TOP_N], ol[:, :TOP_N]


# speedup vs baseline: 1.1964x; 1.1964x over previous
import functools

import jax
import jax.numpy as jnp
from jax import lax
from jax.experimental import pallas as pl
from jax.experimental.pallas import tpu as pltpu
from jax.experimental.pallas import tpu_sc as plsc

SCORE_THRESH = 0.05
NMS_THRESH = 0.45
TOP_N = 100
N = 20000
NPAD = 20480
NCLS = 20
NL = 112
NCH = 1280
NC2 = 80


def _prep_body(priors_ref, deltas_ref, obj_ref, sc_ref, bx_ref, mc_ref):
    pcx = priors_ref[0:1, :]
    pcy = priors_ref[1:2, :]
    pw = priors_ref[2:3, :] + 1e-3
    ph = priors_ref[3:4, :] + 1e-3
    d0 = deltas_ref[0, 0:1, :]
    d1 = deltas_ref[0, 1:2, :]
    d2 = deltas_ref[0, 2:3, :]
    d3 = deltas_ref[0, 3:4, :]
    cx = pcx + d0 * 0.1 * pw
    cy = pcy + d1 * 0.1 * ph
    w = pw * jnp.exp(d2 * 0.2)
    h = ph * jnp.exp(d3 * 0.2)
    x1 = cx - w * 0.5
    y1 = cy - h * 0.5
    x2 = cx + w * 0.5
    y2 = cy + h * 0.5

    n_row = lax.broadcasted_iota(jnp.int32, (1, NPAD), 1)
    vrow = n_row < N
    neg = jnp.float32(-1e30)
    mx = jnp.maximum(
        jnp.maximum(jnp.max(jnp.where(vrow, x1, neg)), jnp.max(jnp.where(vrow, y1, neg))),
        jnp.maximum(jnp.max(jnp.where(vrow, x2, neg)), jnp.max(jnp.where(vrow, y2, neg))),
    )
    mc = mx + 1.0

    o = obj_ref[0]
    m0 = jnp.max(o, axis=0, keepdims=True)
    e = jnp.exp(o - m0)
    p = e / jnp.sum(e, axis=0, keepdims=True)
    S = p[1:21, :]
    valid2d = lax.broadcasted_iota(jnp.int32, (NCLS, NPAD), 1) < N
    sc_ref[0] = jnp.where(valid2d & (S > SCORE_THRESH), S, -1.0)

    bx_ref[0, 0:1, :] = x1
    bx_ref[0, 1:2, :] = y1
    bx_ref[0, 2:3, :] = x2
    bx_ref[0, 3:4, :] = y2
    mc_ref[0] = jnp.full((8, 128), mc, jnp.float32)


def _prep(priors, pred_bbox_deltas, objectness):
    B = pred_bbox_deltas.shape[0]
    pr_t = jnp.pad(priors, ((0, NPAD - N), (0, 0))).T
    dl_t = jnp.pad(pred_bbox_deltas, ((0, 0), (0, NPAD - N), (0, 0))).transpose(0, 2, 1)
    ob_t = jnp.pad(objectness, ((0, 0), (0, NPAD - N), (0, 0))).transpose(0, 2, 1)
    return pl.pallas_call(
        _prep_body,
        grid=(B,),
        in_specs=[
            pl.BlockSpec((4, NPAD), lambda b: (0, 0)),
            pl.BlockSpec((1, 4, NPAD), lambda b: (b, 0, 0)),
            pl.BlockSpec((1, 21, NPAD), lambda b: (b, 0, 0)),
        ],
        out_specs=[
            pl.BlockSpec((1, NCLS, NPAD), lambda b: (b, 0, 0)),
            pl.BlockSpec((1, 4, NPAD), lambda b: (b, 0, 0)),
            pl.BlockSpec((1, 8, 128), lambda b: (b, 0, 0)),
        ],
        out_shape=[
            jax.ShapeDtypeStruct((B, NCLS, NPAD), jnp.float32),
            jax.ShapeDtypeStruct((B, 4, NPAD), jnp.float32),
            jax.ShapeDtypeStruct((B, 8, 128), jnp.float32),
        ],
    )(pr_t, dl_t, ob_t)


def _lane():
    return lax.broadcasted_iota(jnp.int32, (16,), 0)


def _sget(ref, i):
    v = plsc.load_gather(ref, [jnp.full((16,), i, jnp.int32)])
    return jnp.max(v)


def _sgetv(ref, i):
    return plsc.load_gather(ref, [jnp.full((16,), i, jnp.int32)])


def _sget2(ref, i, j):
    v = plsc.load_gather(ref, [jnp.full((16,), i, jnp.int32),
                               jnp.full((16,), j, jnp.int32)])
    return jnp.max(v)


def _set1(ref, i, val):
    base = (i // 16) * 16
    v = ref[pl.ds(base, 16)]
    ref[pl.ds(base, 16)] = jnp.where(_lane() == i - base, val, v)


def _ffs(mask):
    return jnp.max(plsc.all_reduce_ffs(mask))


def _popcnt(mask):
    return jnp.max(plsc.all_reduce_population_count(mask))


def _make_sc_body(cap):
  nchk = (cap + 15) // 16

  def _sc_nms_body(sc_hbm, bx_hbm, mc_hbm, ob_hbm, os_hbm, ol_hbm, ls_hbm, ln_hbm,
                 fl_hbm,
                 S_v, bx_v, L1_v, L2_v, ax1_v, ay1_v, ax2_v, ay2_v, aa_v,
                 ps_v, pn_v, mc_v, fl_v, sa_v, na_v, hd_v, obo_v, oso_v, olo_v):
    f32 = jnp.float32
    cc = lax.axis_index("c")
    ss = lax.axis_index("s")
    bb = ss // 8
    b = cc * 2 + bb
    g = ss % 8

    for k in range(4):
        pltpu.sync_copy(bx_hbm.at[b, k], bx_v.at[pl.ds(k * NPAD, NPAD)])
    pltpu.sync_copy(mc_hbm.at[b, 0], mc_v)
    mc_s = jnp.max(mc_v[pl.ds(0, 16)])
    lane = _lane()

    for slot in range(3):
        for k in range(7):
            ps_v[pl.ds(k * 16, 16)] = jnp.full((16,), -1.0, f32)
            pn_v[pl.ds(k * 16, 16)] = jnp.zeros((16,), jnp.int32)
        valid = (g < 4) | (jnp.int32(slot) < 2)
        cls = jnp.where(g < 4, 3 * g + slot, 12 + 2 * (g - 4) + slot)

        @pl.when(valid)
        def _walk():
            pltpu.sync_copy(sc_hbm.at[b, cls], S_v)
            for k in range(nchk):
                ax1_v[pl.ds(k * 16, 16)] = jnp.full((16,), 1e30, f32)
                ay1_v[pl.ds(k * 16, 16)] = jnp.full((16,), 1e30, f32)
                ax2_v[pl.ds(k * 16, 16)] = jnp.full((16,), 1e30, f32)
                ay2_v[pl.ds(k * 16, 16)] = jnp.full((16,), 1e30, f32)
                aa_v[pl.ds(k * 16, 16)] = jnp.zeros((16,), f32)

            def l1body(t, c):
                base = t * 256
                m = plsc.load_gather(S_v, [lane * 16 + base])
                for l in range(1, 16):
                    m = jnp.maximum(m, plsc.load_gather(S_v, [lane * 16 + (base + l)]))
                L1_v[pl.ds(t * 16, 16)] = m
                return c
            lax.fori_loop(0, NCH // 16, l1body, 0)

            def l2body(t, c):
                base = t * 256
                m = plsc.load_gather(L1_v, [lane * 16 + base])
                for l in range(1, 16):
                    m = jnp.maximum(m, plsc.load_gather(L1_v, [lane * 16 + (base + l)]))
                L2_v[pl.ds(t * 16, 16)] = m
                return c
            lax.fori_loop(0, NC2 // 16, l2body, 0)

            r = L2_v[pl.ds(0, 16)]
            for q in range(1, 5):
                r = jnp.maximum(r, L2_v[pl.ds(q * 16, 16)])
            root0 = jnp.max(r)
            lmc = (cls + 1).astype(f32) * mc_s

            def wcond(carry):
                cnt, pops, fails, root = carry
                return (cnt < cap) & (root > 0.0) & (pops < NPAD) & (fails == 0)

            def wbody(carry):
                cnt, pops, fails, root = carry
                q = jnp.int32(-1)
                for qq in range(5):
                    v = L2_v[pl.ds(qq * 16, 16)]
                    mk = v == root
                    hit = _popcnt(mk) > 0
                    q = jnp.where((q < 0) & hit, qq * 16 + _ffs(mk), q)
                fails = fails + jnp.where(q < 0, 1, 0)
                q = jnp.clip(q, 0, NC2 - 1)
                l1c = L1_v[pl.ds(q * 16, 16)]
                jj = _ffs(l1c == root)
                fails = fails + jnp.where(jj > 15, 1, 0)
                jj = jnp.clip(jj, 0, 15)
                j = q * 16 + jj
                sc_ = S_v[pl.ds(j * 16, 16)]
                ll = _ffs(sc_ == root)
                fails = fails + jnp.where(ll > 15, 1, 0)
                ll = jnp.clip(ll, 0, 15)
                n = j * 16 + ll
                ox1 = _sgetv(bx_v, n) + lmc
                oy1 = _sgetv(bx_v, NPAD + n) + lmc
                ox2 = _sgetv(bx_v, 2 * NPAD + n) + lmc
                oy2 = _sgetv(bx_v, 3 * NPAD + n) + lmc
                a2 = (ox2 - ox1) * (oy2 - oy1)

                rej = jnp.int32(0)
                for k in range(nchk):
                    x1a = ax1_v[pl.ds(k * 16, 16)]
                    y1a = ay1_v[pl.ds(k * 16, 16)]
                    x2a = ax2_v[pl.ds(k * 16, 16)]
                    y2a = ay2_v[pl.ds(k * 16, 16)]
                    aa = aa_v[pl.ds(k * 16, 16)]
                    iw = jnp.maximum(jnp.minimum(x2a, ox2) - jnp.maximum(x1a, ox1), 0.0)
                    ih = jnp.maximum(jnp.minimum(y2a, oy2) - jnp.maximum(y1a, oy1), 0.0)
                    inter = iw * ih
                    iou = inter / (aa + a2 - inter + 1e-9)
                    rej = rej + _popcnt(iou > NMS_THRESH)
                acc = rej == 0

                @pl.when(acc)
                def _accept():
                    _set1(ax1_v, cnt, ox1)
                    _set1(ay1_v, cnt, oy1)
                    _set1(ax2_v, cnt, ox2)
                    _set1(ay2_v, cnt, oy2)
                    _set1(aa_v, cnt, a2)
                    _set1(ps_v, cnt, root)
                    _set1(pn_v, cnt, n)

                sc2 = jnp.where(lane == ll, -1.0, sc_)
                S_v[pl.ds(j * 16, 16)] = sc2
                l1n = jnp.max(sc2)
                l1c2 = jnp.where(lane == jj, l1n, l1c)
                L1_v[pl.ds(q * 16, 16)] = l1c2
                qq16 = (q // 16) * 16
                l2c = L2_v[pl.ds(qq16, 16)]
                l2c2 = jnp.where(lane == q - qq16, jnp.max(l1c2), l2c)
                L2_v[pl.ds(qq16, 16)] = l2c2
                r = L2_v[pl.ds(0, 16)]
                for qq in range(1, 5):
                    r = jnp.maximum(r, L2_v[pl.ds(qq * 16, 16)])
                return (jnp.where(acc, cnt + 1, cnt), pops + 1, fails, jnp.max(r))

            cntf, _, _, rootf = lax.while_loop(
                wcond, wbody, (jnp.int32(0), jnp.int32(0), jnp.int32(0), root0))

            @pl.when((cntf == cap) & (rootf > 0.0))
            def _mark_truncated():
                _set1(ps_v, cap, jnp.float32(-2.0))

        li = g * 3 + slot
        pltpu.sync_copy(ps_v, ls_hbm.at[b, li])
        pltpu.sync_copy(pn_v, ln_hbm.at[b, li])

    plsc.subcore_barrier()

    @pl.when(g == 0)
    def _merge():
        for l in range(24, 32):
            for k in range(7):
                sa_v[l, pl.ds(k * 16, 16)] = jnp.full((16,), -1.0, f32)
        pltpu.sync_copy(ls_hbm.at[b, pl.ds(0, 24)], sa_v.at[pl.ds(0, 24)])
        pltpu.sync_copy(ln_hbm.at[b, pl.ds(0, 24)], na_v.at[pl.ds(0, 24)])
        hd_v[pl.ds(0, 16)] = jnp.zeros((16,), jnp.int32)
        hd_v[pl.ds(16, 16)] = jnp.zeros((16,), jnp.int32)
        for k in range(7):
            oso_v[pl.ds(k * 16, 16)] = jnp.zeros((16,), f32)
            olo_v[pl.ds(k * 16, 16)] = jnp.zeros((16,), jnp.int32)
        for k in range(28):
            obo_v[pl.ds(k * 16, 16)] = jnp.zeros((16,), f32)

        def heads_root():
            h1 = hd_v[pl.ds(0, 16)]
            h2 = hd_v[pl.ds(16, 16)]
            sv1 = plsc.load_gather(sa_v, [lane, h1])
            sv2 = plsc.load_gather(sa_v, [lane + 16, h2])
            return sv1, sv2, jnp.max(jnp.maximum(sv1, sv2))

        _, _, root0 = heads_root()

        def mcond(carry):
            t, root = carry
            return (t < TOP_N) & (root > 0.0)

        def mbody(carry):
            t, root = carry
            h1 = hd_v[pl.ds(0, 16)]
            h2 = hd_v[pl.ds(16, 16)]
            sv1 = plsc.load_gather(sa_v, [lane, h1])
            sv2 = plsc.load_gather(sa_v, [lane + 16, h2])
            mk1 = sv1 == root
            lsel = jnp.where(_popcnt(mk1) > 0, _ffs(mk1),
                             16 + jnp.clip(_ffs(sv2 == root), 0, 15))
            lsel = jnp.clip(lsel, 0, 31)
            h = jnp.max(jnp.maximum(jnp.where(lane == lsel, h1, 0),
                                    jnp.where(lane + 16 == lsel, h2, 0)))
            n = _sget2(na_v, lsel, h)
            hd_v[pl.ds(0, 16)] = jnp.where(lane == lsel, h + 1, h1)
            hd_v[pl.ds(16, 16)] = jnp.where(lane + 16 == lsel, h + 1, h2)
            g2 = lsel // 3
            sl = lsel - g2 * 3
            clsm = jnp.where(g2 < 4, 3 * g2 + sl, 12 + 2 * (g2 - 4) + sl)
            _set1(oso_v, t, root)
            _set1(olo_v, t, clsm + 1)
            for k in range(4):
                v = _sget(bx_v, k * NPAD + n)
                _set1(obo_v, k * NL + t, jnp.minimum(jnp.maximum(v, 0.0), 1.0))
            h1 = hd_v[pl.ds(0, 16)]
            h2 = hd_v[pl.ds(16, 16)]
            sv1 = plsc.load_gather(sa_v, [lane, h1])
            sv2 = plsc.load_gather(sa_v, [lane + 16, h2])
            return t + 1, jnp.max(jnp.maximum(sv1, sv2))

        lax.while_loop(mcond, mbody, (jnp.int32(0), root0))
        h1 = hd_v[pl.ds(0, 16)]
        h2 = hd_v[pl.ds(16, 16)]
        sv1 = plsc.load_gather(sa_v, [lane, h1])
        sv2 = plsc.load_gather(sa_v, [lane + 16, h2])
        vio = _popcnt(sv1 == -2.0) + _popcnt(sv2 == -2.0)
        fl_v[pl.ds(0, 16)] = jnp.where(lane == 0, vio, 0)
        pltpu.sync_copy(fl_v, fl_hbm.at[b])
        pltpu.sync_copy(obo_v, ob_hbm.at[b])
        pltpu.sync_copy(oso_v, os_hbm.at[b])
        pltpu.sync_copy(olo_v, ol_hbm.at[b])

  return _sc_nms_body


def _sc_nms(scores, boxes, mcs, cap):
    B = scores.shape[0]
    f32, i32 = jnp.float32, jnp.int32
    mesh = plsc.VectorSubcoreMesh(core_axis_name="c", subcore_axis_name="s")
    fn = functools.partial(
        pl.kernel, mesh=mesh,
        compiler_params=pltpu.CompilerParams(needs_layout_passes=False),
        out_type=[
            jax.ShapeDtypeStruct((B, 4 * NL), f32),
            jax.ShapeDtypeStruct((B, NL), f32),
            jax.ShapeDtypeStruct((B, NL), i32),
            jax.ShapeDtypeStruct((B, 32, NL), f32),
            jax.ShapeDtypeStruct((B, 32, NL), i32),
            jax.ShapeDtypeStruct((B, 16), i32),
        ],
        scratch_types=[
            pltpu.VMEM((NPAD,), f32),
            pltpu.VMEM((4 * NPAD,), f32),
            pltpu.VMEM((NCH,), f32),
            pltpu.VMEM((NC2,), f32),
            pltpu.VMEM((NL,), f32),
            pltpu.VMEM((NL,), f32),
            pltpu.VMEM((NL,), f32),
            pltpu.VMEM((NL,), f32),
            pltpu.VMEM((NL,), f32),
            pltpu.VMEM((NL,), f32),
            pltpu.VMEM((NL,), i32),
            pltpu.VMEM((128,), f32),
            pltpu.VMEM((16,), i32),
            pltpu.VMEM((32, NL), f32),
            pltpu.VMEM((32, NL), i32),
            pltpu.VMEM((32,), i32),
            pltpu.VMEM((4 * NL,), f32),
            pltpu.VMEM((NL,), f32),
            pltpu.VMEM((NL,), i32),
        ],
    )(_make_sc_body(cap))
    return fn(scores, boxes, mcs)


FAST_CAP = 32


def kernel(priors, pred_bbox_deltas, objectness):
    B = pred_bbox_deltas.shape[0]
    scores, boxes, mcs = _prep(priors, pred_bbox_deltas, objectness)
    fast = _sc_nms(scores, boxes, mcs, FAST_CAP)
    ok = jnp.sum(fast[5]) == 0

    def _use_fast(ops):
        return ops[0][:3]

    def _full(ops):
        return _sc_nms(*ops[1], TOP_N)[:3]

    ob, osc, ol = lax.cond(ok, _use_fast, _full, (fast, (scores, boxes, mcs)))
    boxes_out = ob.reshape(B, 4, NL)[:, :, :TOP_N].transpose(0, 2, 1)
    return boxes_out, osc[:, :TOP_N], ol[:, :TOP_N]
